# baseline (speedup 1.0000x reference)
import functools

import jax
import jax.numpy as jnp
from jax import lax
from jax.experimental import pallas as pl
from jax.experimental.pallas import tpu as pltpu
from jax.experimental.pallas import tpu_sc as plsc


def kernel(user, memory, user_table, item_table):
    B, MEM = memory.shape
    D = item_table.shape[1]
    info = plsc.get_sparse_core_info()
    NC, NS = info.num_cores, info.num_subcores
    NW = NC * NS
    per_w = B // NW
    NSLOT = MEM + 1

    mesh = plsc.VectorSubcoreMesh(core_axis_name="c", subcore_axis_name="s")

    @functools.partial(
        pl.kernel,
        mesh=mesh,
        out_type=jax.ShapeDtypeStruct((B, NSLOT, D), jnp.float32),
        scratch_types=[
            pltpu.VMEM((per_w, MEM), jnp.int32),
            pltpu.VMEM((per_w + 16,), jnp.int32),
            pltpu.VMEM((NSLOT * 8, D), jnp.float32),
            pltpu.VMEM((1, NSLOT, D), jnp.float32),
            pltpu.SemaphoreType.DMA,
        ],
    )
    def k(user_hbm, memory_hbm, utab_hbm, itab_hbm, out_hbm,
          midx, uidx, bank, obuf, gsem):
        wid = lax.axis_index("s") * NC + lax.axis_index("c")
        wbase = wid * per_w

        def idx_scalar(row, i):
            start = i if i <= MEM - 16 else MEM - 16
            return midx[row, pl.ds(start, 16)][i - start]

        pltpu.sync_copy(memory_hbm.at[pl.ds(wbase, per_w)], midx)
        pltpu.sync_copy(user_hbm.at[pl.ds(wbase, per_w)],
                        uidx.at[pl.ds(0, per_w)])

        def body(row, carry):
            cps = []
            for i in range(MEM):
                s = idx_scalar(row, i)
                base8 = pl.multiple_of((s // 8) * 8, 8)
                cps.append(pltpu.async_copy(
                    itab_hbm.at[pl.ds(base8, 8)],
                    bank.at[pl.ds(i * 8, 8)], gsem))
            su = uidx[pl.ds(row, 16)][0]
            ubase8 = pl.multiple_of((su // 8) * 8, 8)
            cps.append(pltpu.async_copy(
                utab_hbm.at[pl.ds(ubase8, 8)],
                bank.at[pl.ds(MEM * 8, 8)], gsem))
            for c in cps:
                c.wait()
            for i in range(MEM):
                s = idx_scalar(row, i)
                r = s - (s // 8) * 8
                for d0 in range(0, D, 16):
                    obuf[0, i, pl.ds(d0, 16)] = bank[i * 8 + r, pl.ds(d0, 16)]
            ru = su - (su // 8) * 8
            for d0 in range(0, D, 16):
                obuf[0, MEM, pl.ds(d0, 16)] = bank[MEM * 8 + ru,
                                                   pl.ds(d0, 16)]
            pltpu.sync_copy(obuf, out_hbm.at[pl.ds(wbase + row, 1)])
            return carry

        lax.fori_loop(0, per_w, body, 0)

    out3 = k(user, memory, user_table, item_table)
    return out3.reshape(B, NSLOT * D)


# speedup vs baseline: 1.1044x; 1.1044x over previous
import functools

import jax
import jax.numpy as jnp
from jax import lax
from jax.experimental import pallas as pl
from jax.experimental.pallas import tpu as pltpu
from jax.experimental.pallas import tpu_sc as plsc


def kernel(user, memory, user_table, item_table):
    B, MEM = memory.shape
    D = item_table.shape[1]
    info = plsc.get_sparse_core_info()
    NC, NS = info.num_cores, info.num_subcores
    NW = NC * NS
    per_w = B // NW
    NSLOT = MEM + 1

    mesh = plsc.VectorSubcoreMesh(core_axis_name="c", subcore_axis_name="s")

    @functools.partial(
        pl.kernel,
        mesh=mesh,
        out_type=jax.ShapeDtypeStruct((B, NSLOT, D), jnp.float32),
        scratch_types=[
            pltpu.VMEM((per_w, MEM), jnp.int32),
            pltpu.VMEM((per_w + 16,), jnp.int32),
            pltpu.VMEM((NSLOT * 8, D), jnp.float32),
            pltpu.VMEM((NSLOT * 8, D), jnp.float32),
            pltpu.VMEM((1, NSLOT, D), jnp.float32),
            pltpu.SemaphoreType.DMA,
            pltpu.SemaphoreType.DMA,
        ],
    )
    def k(user_hbm, memory_hbm, utab_hbm, itab_hbm, out_hbm,
          midx, uidx, bankA, bankB, obuf, gsemA, gsemB):
        wid = lax.axis_index("s") * NC + lax.axis_index("c")
        wbase = wid * per_w
        banks = ((bankA, gsemA), (bankB, gsemB))

        def idx_scalar(row, i):
            start = i if i <= MEM - 16 else MEM - 16
            return midx[row, pl.ds(start, 16)][i - start]

        def fire_row(row, bank, gsem):
            for i in range(MEM):
                s = idx_scalar(row, i)
                base8 = pl.multiple_of((s // 8) * 8, 8)
                pltpu.async_copy(itab_hbm.at[pl.ds(base8, 8)],
                                 bank.at[pl.ds(i * 8, 8)], gsem)
            su = uidx[pl.ds(row, 16)][0]
            ubase8 = pl.multiple_of((su // 8) * 8, 8)
            pltpu.async_copy(utab_hbm.at[pl.ds(ubase8, 8)],
                             bank.at[pl.ds(MEM * 8, 8)], gsem)

        def drain_row(bank, gsem):
            for i in range(NSLOT):
                pltpu.make_async_copy(itab_hbm.at[pl.ds(0, 8)],
                                      bank.at[pl.ds(i * 8, 8)], gsem).wait()

        def select_out(row, bank):
            for i in range(MEM):
                s = idx_scalar(row, i)
                r = s - (s // 8) * 8
                for d0 in range(0, D, 16):
                    obuf[0, i, pl.ds(d0, 16)] = bank[i * 8 + r, pl.ds(d0, 16)]
            su = uidx[pl.ds(row, 16)][0]
            ru = su - (su // 8) * 8
            for d0 in range(0, D, 16):
                obuf[0, MEM, pl.ds(d0, 16)] = bank[MEM * 8 + ru,
                                                   pl.ds(d0, 16)]
            pltpu.sync_copy(obuf, out_hbm.at[pl.ds(wbase + row, 1)])

        pltpu.sync_copy(memory_hbm.at[pl.ds(wbase, per_w)], midx)
        pltpu.sync_copy(user_hbm.at[pl.ds(wbase, per_w)],
                        uidx.at[pl.ds(0, per_w)])
        fire_row(0, bankA, gsemA)
        fire_row(1, bankB, gsemB)

        def body(t, carry):
            row = 2 * t
            for par, (bank, gsem) in enumerate(banks):
                drain_row(bank, gsem)
                select_out(row + par, bank)
                fire_row(row + 2 + par, bank, gsem)
            return carry

        lax.fori_loop(0, per_w // 2 - 1, body, 0)

        for par, (bank, gsem) in enumerate(banks):
            drain_row(bank, gsem)
            select_out(per_w - 2 + par, bank)

    out3 = k(user, memory, user_table, item_table)
    return out3.reshape(B, NSLOT * D)


# speedup vs baseline: 1.3179x; 1.1933x over previous
import functools

import jax
import jax.numpy as jnp
from jax import lax
from jax.experimental import pallas as pl
from jax.experimental.pallas import tpu as pltpu
from jax.experimental.pallas import tpu_sc as plsc

MPAD = 80


def kernel(user, memory, user_table, item_table):
    B, MEM = memory.shape
    D = item_table.shape[1]
    info = plsc.get_sparse_core_info()
    NC, NS = info.num_cores, info.num_subcores
    NW = NC * NS
    per_w = B // NW
    NSLOT = MEM + 1

    memory_p = jnp.pad(memory, ((0, 0), (0, MPAD - MEM)))

    mesh = plsc.VectorSubcoreMesh(core_axis_name="c", subcore_axis_name="s")

    @functools.partial(
        pl.kernel,
        mesh=mesh,
        out_type=jax.ShapeDtypeStruct((B, NSLOT, D), jnp.float32),
        scratch_types=[
            pltpu.VMEM((per_w, MPAD), jnp.int32),
            pltpu.VMEM((per_w + 16,), jnp.int32),
            pltpu.VMEM((NSLOT * 8, D), jnp.float32),
            pltpu.VMEM((NSLOT * 8, D), jnp.float32),
            pltpu.VMEM((NSLOT, D), jnp.float32),
            pltpu.SemaphoreType.DMA,
            pltpu.SemaphoreType.DMA,
        ],
    )
    def k(user_hbm, memory_hbm, utab_hbm, itab_hbm, out_hbm,
          midx, uidx, bankA, bankB, obuf, gsemA, gsemB):
        wid = lax.axis_index("s") * NC + lax.axis_index("c")
        wbase = wid * per_w
        banks = ((bankA, gsemA), (bankB, gsemB))

        def fire_row(row, bank, gsem):
            def fire_one(i, carry):
                s = midx[row, pl.ds(i, 16)][0]
                base8 = pl.multiple_of((s // 8) * 8, 8)
                pltpu.async_copy(itab_hbm.at[pl.ds(base8, 8)],
                                 bank.at[pl.ds(i * 8, 8)], gsem)
                return carry

            lax.fori_loop(0, MEM, fire_one, 0)
            su = uidx[pl.ds(row, 16)][0]
            ubase8 = pl.multiple_of((su // 8) * 8, 8)
            pltpu.async_copy(utab_hbm.at[pl.ds(ubase8, 8)],
                             bank.at[pl.ds(MEM * 8, 8)], gsem)

        def drain_bank(bank, gsem):
            pltpu.make_async_copy(itab_hbm.at[pl.ds(0, NSLOT * 8)],
                                  bank, gsem).wait()

        def select_out(row, bank):
            def sel_one(i, carry):
                s = midx[row, pl.ds(i, 16)][0]
                r = s - (s // 8) * 8
                for d0 in range(0, D, 16):
                    obuf[i, pl.ds(d0, 16)] = bank[i * 8 + r, pl.ds(d0, 16)]
                return carry

            lax.fori_loop(0, MEM, sel_one, 0)
            su = uidx[pl.ds(row, 16)][0]
            ru = su - (su // 8) * 8
            for d0 in range(0, D, 16):
                obuf[MEM, pl.ds(d0, 16)] = bank[MEM * 8 + ru, pl.ds(d0, 16)]
            pltpu.sync_copy(obuf, out_hbm.at[wbase + row])

        pltpu.sync_copy(memory_hbm.at[pl.ds(wbase, per_w)], midx)
        pltpu.sync_copy(user_hbm.at[pl.ds(wbase, per_w)],
                        uidx.at[pl.ds(0, per_w)])
        fire_row(0, bankA, gsemA)
        fire_row(1, bankB, gsemB)

        def body(t, carry):
            row = 2 * t
            for par, (bank, gsem) in enumerate(banks):
                drain_bank(bank, gsem)
                select_out(row + par, bank)
                fire_row(row + 2 + par, bank, gsem)
            return carry

        lax.fori_loop(0, per_w // 2 - 1, body, 0)

        for par, (bank, gsem) in enumerate(banks):
            drain_bank(bank, gsem)
            select_out(per_w - 2 + par, bank)

    out3 = k(user, memory_p, user_table, item_table)
    return out3.reshape(B, NSLOT * D)


# speedup vs baseline: 1.3226x; 1.0036x over previous
import functools

import jax
import jax.numpy as jnp
from jax import lax
from jax.experimental import pallas as pl
from jax.experimental.pallas import tpu as pltpu
from jax.experimental.pallas import tpu_sc as plsc

MPAD = 64


def kernel(user, memory, user_table, item_table):
    B, MEM = memory.shape
    D = item_table.shape[1]
    info = plsc.get_sparse_core_info()
    NC, NS = info.num_cores, info.num_subcores
    NW = NC * NS
    per_w = B // NW
    NSLOT = MEM + 1

    memory_p = jnp.pad(memory, ((0, 0), (0, MPAD - MEM)))

    mesh = plsc.VectorSubcoreMesh(core_axis_name="c", subcore_axis_name="s")

    @functools.partial(
        pl.kernel,
        mesh=mesh,
        out_type=jax.ShapeDtypeStruct((B, NSLOT, D), jnp.float32),
        scratch_types=[
            pltpu.VMEM((per_w, MPAD), jnp.int32),
            pltpu.VMEM((per_w + 16,), jnp.int32),
            pltpu.VMEM((NSLOT * 8, D), jnp.float32),
            pltpu.VMEM((NSLOT * 8, D), jnp.float32),
            pltpu.VMEM((NSLOT, D), jnp.float32),
            pltpu.SemaphoreType.DMA,
            pltpu.SemaphoreType.DMA,
        ],
    )
    def k(user_hbm, memory_hbm, utab_hbm, itab_hbm, out_hbm,
          midx, uidx, bankA, bankB, obuf, gsemA, gsemB):
        wid = lax.axis_index("s") * NC + lax.axis_index("c")
        wbase = wid * per_w
        banks = ((bankA, gsemA), (bankB, gsemB))

        def fire_row(row, bank, gsem):
            for k0 in range(0, MEM, 16):
                cnt = min(16, MEM - k0)
                vs = midx[row, pl.ds(k0, 16)]
                vb = (vs >> 3) << 3
                for j in range(cnt):
                    s8 = pl.multiple_of(vb[j], 8)
                    pltpu.async_copy(itab_hbm.at[pl.ds(s8, 8)],
                                     bank.at[pl.ds((k0 + j) * 8, 8)], gsem)
            vu = uidx[pl.ds(row, 16)]
            su8 = pl.multiple_of(((vu >> 3) << 3)[0], 8)
            pltpu.async_copy(utab_hbm.at[pl.ds(su8, 8)],
                             bank.at[pl.ds(MEM * 8, 8)], gsem)

        def drain_bank(bank, gsem):
            pltpu.make_async_copy(itab_hbm.at[pl.ds(0, NSLOT * 8)],
                                  bank, gsem).wait()

        def select_out(row, bank):
            for k0 in range(0, MEM, 16):
                cnt = min(16, MEM - k0)
                vs = midx[row, pl.ds(k0, 16)]
                vr = vs - ((vs >> 3) << 3)
                for j in range(cnt):
                    r = vr[j]
                    for d0 in range(0, D, 16):
                        obuf[k0 + j, pl.ds(d0, 16)] = \
                            bank[(k0 + j) * 8 + r, pl.ds(d0, 16)]
            vu = uidx[pl.ds(row, 16)]
            rr = (vu - ((vu >> 3) << 3))[0]
            for d0 in range(0, D, 16):
                obuf[MEM, pl.ds(d0, 16)] = bank[MEM * 8 + rr, pl.ds(d0, 16)]
            pltpu.sync_copy(obuf, out_hbm.at[wbase + row])

        pltpu.sync_copy(memory_hbm.at[pl.ds(wbase, per_w)], midx)
        pltpu.sync_copy(user_hbm.at[pl.ds(wbase, per_w)],
                        uidx.at[pl.ds(0, per_w)])
        fire_row(0, bankA, gsemA)
        fire_row(1, bankB, gsemB)

        def body(t, carry):
            row = 2 * t
            for par, (bank, gsem) in enumerate(banks):
                drain_bank(bank, gsem)
                select_out(row + par, bank)
                fire_row(row + 2 + par, bank, gsem)
            return carry

        lax.fori_loop(0, per_w // 2 - 1, body, 0)

        for par, (bank, gsem) in enumerate(banks):
            drain_bank(bank, gsem)
            select_out(per_w - 2 + par, bank)

    out3 = k(user, memory_p, user_table, item_table)
    return out3.reshape(B, NSLOT * D)
